# baseline (speedup 1.0000x reference)
import functools

import jax
import jax.numpy as jnp
from jax import lax
from jax.experimental import pallas as pl
from jax.experimental.pallas import tpu as pltpu
from jax.experimental.pallas import tpu_sc as plsc

NUM_ENTITIES = 1000000
NUM_RELATIONS = 1000
DIM = 64
HALF = DIM // 2
B = 16384

NC = 2
NS = 16
NW = NC * NS
BPW = B // NW

_RE = 32768
_RQ = _RE // 4
_RE_BLOCKS = -(-NUM_ENTITIES // _RE)
NQUAD = _RE_BLOCKS * _RQ
_LB = _RE.bit_length() - 1


def _repack_body(rt_ref, it_ref, c_ref):
    eye = jnp.eye(2 * DIM, dtype=jnp.float32)
    dn = (((0,), (0,)), ((), ()))
    q = _RQ
    rt = rt_ref[...]
    it = it_ref[...]
    a_lo = jnp.concatenate(
        [rt[:, 0:q], it[:, 0:q], rt[:, 2 * q : 3 * q], it[:, 2 * q : 3 * q]],
        axis=0,
    )
    a_hi = jnp.concatenate(
        [rt[:, q : 2 * q], it[:, q : 2 * q], rt[:, 3 * q :], it[:, 3 * q :]],
        axis=0,
    )

    pid = pl.program_id(0)
    last = pid == _RE_BLOCKS - 1

    def pack(lo_f32, hi_f32):
        lo = lax.bitcast_convert_type(lo_f32, jnp.uint32)
        hi = lax.bitcast_convert_type(hi_f32, jnp.uint32)
        lo16 = (lo + 0x8000) >> 16
        hi16 = (hi + 0x8000) & jnp.uint32(0xFFFF0000)
        return lax.bitcast_convert_type(lo16 | hi16, jnp.int32)

    @pl.when(jnp.logical_not(last))
    def _():
        tl = lax.dot_general(a_lo, eye, dn, preferred_element_type=jnp.float32)
        th = lax.dot_general(a_hi, eye, dn, preferred_element_type=jnp.float32)
        c_ref[...] = pack(tl, th)

    @pl.when(last)
    def _():
        ko = jnp.where(
            lax.broadcasted_iota(jnp.int32, (2 * DIM, q), 0) >= DIM, 2 * q, 0
        )
        ent = pid * _RE + ko + lax.broadcasted_iota(jnp.int32, (2 * DIM, q), 1)
        a_lo_m = jnp.where(ent < NUM_ENTITIES, a_lo, 0.0)
        a_hi_m = jnp.where(ent + q < NUM_ENTITIES, a_hi, 0.0)
        tl = lax.dot_general(a_lo_m, eye, dn, preferred_element_type=jnp.float32)
        th = lax.dot_general(a_hi_m, eye, dn, preferred_element_type=jnp.float32)
        c_ref[...] = pack(tl, th)


def _repack_call(ret, imt):
    return pl.pallas_call(
        _repack_body,
        grid=(_RE_BLOCKS,),
        in_specs=[
            pl.BlockSpec((HALF, _RE), lambda i: (0, i)),
            pl.BlockSpec((HALF, _RE), lambda i: (0, i)),
        ],
        out_specs=pl.BlockSpec((_RQ, 2 * DIM), lambda i: (i, 0)),
        out_shape=jax.ShapeDtypeStruct((NQUAD, 2 * DIM), jnp.int32),
        compiler_params=pltpu.CompilerParams(fuse_transposed_lhs_in_matmul=True),
        name="tc_repack",
    )(ret, imt)


def _gather_body(c_hbm, srch_hbm, tgth_hbm, out_hbm, idx_v, rows_v, sem):
    wid = lax.axis_index("s") * NC + lax.axis_index("c")
    base = wid * BPW
    pltpu.sync_copy(srch_hbm.at[pl.ds(base, BPW)], idx_v)
    pltpu.async_copy(c_hbm.at[idx_v], rows_v, sem).wait()
    pltpu.sync_copy(rows_v, out_hbm.at[0, pl.ds(base, BPW)])
    pltpu.sync_copy(tgth_hbm.at[pl.ds(base, BPW)], idx_v)
    pltpu.async_copy(c_hbm.at[idx_v], rows_v, sem).wait()
    pltpu.sync_copy(rows_v, out_hbm.at[1, pl.ds(base, BPW)])


@functools.cache
def _gather_call():
    return pl.kernel(
        _gather_body,
        out_type=jax.ShapeDtypeStruct((2, B, 2 * DIM), jnp.int32),
        mesh=plsc.VectorSubcoreMesh(core_axis_name="c", subcore_axis_name="s"),
        scratch_types=[
            pltpu.VMEM((BPW,), jnp.int32),
            pltpu.VMEM((BPW, 2 * DIM), jnp.int32),
            pltpu.SemaphoreType.DMA,
        ],
        name="sc_gather2",
    )


_BS = 4096
_INV_SQRT2 = 0.7071067811865476


def _unpack_half(grow, lane_bit, half_bit):
    w = jnp.where(lane_bit > 0, grow[:, DIM:], grow[:, :DIM])
    bits = jnp.where(
        half_bit > 0,
        w & jnp.int32(-65536),
        w << 16,
    )
    return lax.bitcast_convert_type(bits, jnp.float32)


def _mlp_body(g_ref, ps_ref, pt_ref, w1_ref, b1_ref, w2_ref, b2_ref, o_ref):
    hs = _unpack_half(g_ref[0], ps_ref[...] & 2, ps_ref[...] & 1)
    ht = _unpack_half(g_ref[1], pt_ref[...] & 2, pt_ref[...] & 1)
    h1 = (
        jnp.dot(hs, w1_ref[0], preferred_element_type=jnp.float32)
        + jnp.dot(ht, w1_ref[1], preferred_element_type=jnp.float32)
        + b1_ref[...]
    )
    h1 = 0.5 * h1 * (1.0 + lax.erf(h1 * _INV_SQRT2))
    o_ref[...] = jnp.dot(h1, w2_ref[...], preferred_element_type=jnp.float32) + b2_ref[...]


def _mlp_call(g, psrc, ptgt, w1, b1, w2, b2):
    return pl.pallas_call(
        _mlp_body,
        grid=(B // _BS,),
        in_specs=[
            pl.BlockSpec((2, _BS, 2 * DIM), lambda i: (0, i, 0)),
            pl.BlockSpec((_BS, 1), lambda i: (i, 0)),
            pl.BlockSpec((_BS, 1), lambda i: (i, 0)),
            pl.BlockSpec((2, DIM, DIM), lambda i: (0, 0, 0)),
            pl.BlockSpec((1, DIM), lambda i: (0, 0)),
            pl.BlockSpec((DIM, NUM_RELATIONS), lambda i: (0, 0)),
            pl.BlockSpec((1, NUM_RELATIONS), lambda i: (0, 0)),
        ],
        out_specs=pl.BlockSpec((_BS, NUM_RELATIONS), lambda i: (i, 0)),
        out_shape=jax.ShapeDtypeStruct((B, NUM_RELATIONS), jnp.float32),
        name="tc_mlp",
    )(g, psrc, ptgt, w1, b1, w2, b2)


def _row_slot(e):
    return ((e >> _LB) << (_LB - 2)) + (e & (_RQ - 1)), (e >> (_LB - 2)) & 3


@jax.jit
def kernel(src, tgt, entity_re, entity_im, W1, b1, W2, b2):
    src = src.astype(jnp.int32)
    tgt = tgt.astype(jnp.int32)
    c = _repack_call(entity_re.T, entity_im.T)
    src_row, src_slot = _row_slot(src)
    tgt_row, tgt_slot = _row_slot(tgt)
    g = _gather_call()(c, src_row, tgt_row)
    return _mlp_call(
        g,
        src_slot.reshape(B, 1),
        tgt_slot.reshape(B, 1),
        W1.reshape(2, DIM, DIM),
        b1.reshape(1, DIM),
        W2,
        b2.reshape(1, NUM_RELATIONS),
    )
